# baseline (speedup 1.0000x reference)
import functools

import jax
import jax.numpy as jnp
from jax import lax
from jax.experimental import pallas as pl
from jax.experimental.pallas import tpu as pltpu
from jax.experimental.pallas import tpu_sc as plsc

N = 10000
E = 320000
D = 128
G = 64
WROW = 144
ACCW = 128
NPAD = 10240
NTILES = 32
CH = 64
NCHUNK = 160
EPT = NCHUNK * CH
EPAD = EPT * NTILES
BN = 1000
NBLK = N // BN


def _edge_body(hpad_hbm, as_hbm, ad_hbm, pk_hbm, acc_out, s_out,
               pk_all, rg0, rg1, rs0, rs1,
               av0, av1, dv0, dv1,
               sc0, sc1, dg0, dg1, ds0, ds1,
               w0, w1, zb, acc_sh, s_sh,
               gs0, gs1, ss0, ss1):
    c = lax.axis_index("c")
    s = lax.axis_index("s")
    wid = s * 2 + c
    base_e = wid * EPT

    pltpu.sync_copy(pk_hbm.at[pl.ds(base_e, EPT)], pk_all)

    for b in range(16):
        for j in range(ACCW // 16):
            rs0[b, pl.ds(j * 16, 16)] = jnp.zeros((16,), jnp.float32)
    for j in range(8):
        zb[pl.ds(j * 16, 16)] = jnp.zeros((16,), jnp.float32)

    def zstep(k, _):
        pltpu.sync_copy(rs0.at[pl.ds(0, 16)],
                        acc_sh.at[pl.ds(s * 640 + k * 16, 16)])
        return 0
    lax.fori_loop(0, 40, zstep, 0)

    def zstep2(k, _):
        pltpu.sync_copy(zb, s_sh.at[pl.ds(s * 640 + k * 128, 128)])
        return 0
    lax.fori_loop(0, 5, zstep2, 0)
    plsc.subcore_barrier()

    def stage_idx(scur, dgc, i):
        for j in range(CH // 16):
            pk = pk_all[pl.ds(i * CH + j * 16, 16)]
            scur[pl.ds(j * 16, 16)] = jnp.bitwise_and(pk, 16383)
            dgc[pl.ds(j * 16, 16)] = jnp.right_shift(pk, 14)

    def start_gathers(scur, dgc, rg, av, dv, gsem):
        pltpu.async_copy(hpad_hbm.at[scur], rg, gsem)
        pltpu.async_copy(as_hbm.at[scur], av, gsem)
        pltpu.async_copy(ad_hbm.at[dgc], dv, gsem)

    def wait_gathers(scur, dgc, rg, av, dv, gsem):
        pltpu.make_async_copy(hpad_hbm.at[scur], rg, gsem).wait()
        pltpu.make_async_copy(as_hbm.at[scur], av, gsem).wait()
        pltpu.make_async_copy(ad_hbm.at[dgc], dv, gsem).wait()

    def compute_w(av, dv, wb):
        for j in range(CH // 16):
            e = av[pl.ds(j * 16, 16)] + dv[pl.ds(j * 16, 16)]
            e = jnp.maximum(e, e * 0.2)
            wb[pl.ds(j * 16, 16)] = jnp.exp(e)

    def snap_dst(dgc, dsc):
        for j in range(CH // 16):
            dsc[pl.ds(j * 16, 16)] = dgc[pl.ds(j * 16, 16)]

    def scale(rg, rs, wb):
        def grp(gi, _):
            wv = wb[pl.ds(gi * 16, 16)]
            for l in range(16):
                wl = wv[l]
                b = gi * 16 + l
                for j in range(ACCW // 16):
                    rs[b, pl.ds(j * 16, 16)] = rg[b, pl.ds(j * 16, 16)] * wl
            return 0
        lax.fori_loop(0, CH // 16, grp, 0)

    def start_scatters(rs, wb, dsc, ssem):
        pltpu.async_copy(rs, acc_sh.at[dsc], ssem, add=True)
        pltpu.async_copy(wb, s_sh.at[dsc], ssem, add=True)

    def wait_scatters(rs, wb, dsc, ssem):
        pltpu.make_async_copy(rs, acc_sh.at[dsc], ssem).wait()
        pltpu.make_async_copy(wb, s_sh.at[dsc], ssem).wait()

    bufs = ((rg0, rs0, av0, dv0, sc0, dg0, ds0, w0, gs0, ss0),
            (rg1, rs1, av1, dv1, sc1, dg1, ds1, w1, gs1, ss1))

    for p in range(2):
        rg, rs, av, dv, scur, dgc, dsc, wb, gsem, ssem = bufs[p]
        stage_idx(scur, dgc, p)
        start_gathers(scur, dgc, rg, av, dv, gsem)

    for p in range(2):
        rg, rs, av, dv, scur, dgc, dsc, wb, gsem, ssem = bufs[p]
        wait_gathers(scur, dgc, rg, av, dv, gsem)
        compute_w(av, dv, wb)
        snap_dst(dgc, dsc)
        scale(rg, rs, wb)
        start_scatters(rs, wb, dsc, ssem)
        stage_idx(scur, dgc, p + 2)
        start_gathers(scur, dgc, rg, av, dv, gsem)

    def steady(gstep, _):
        for p in range(2):
            rg, rs, av, dv, scur, dgc, dsc, wb, gsem, ssem = bufs[p]
            i = gstep * 2 + p
            wait_scatters(rs, wb, dsc, ssem)
            wait_gathers(scur, dgc, rg, av, dv, gsem)
            compute_w(av, dv, wb)
            snap_dst(dgc, dsc)
            scale(rg, rs, wb)
            start_scatters(rs, wb, dsc, ssem)

            @pl.when(i < NCHUNK - 2)
            def _():
                stage_idx(scur, dgc, i + 2)
                start_gathers(scur, dgc, rg, av, dv, gsem)
        return 0
    lax.fori_loop(1, NCHUNK // 2, steady, 0)

    for p in range(2):
        rg, rs, av, dv, scur, dgc, dsc, wb, gsem, ssem = bufs[p]
        wait_scatters(rs, wb, dsc, ssem)
    plsc.subcore_barrier()

    r0 = s * 624
    pltpu.sync_copy(acc_sh.at[pl.ds(r0, 640)], acc_out.at[c, pl.ds(r0, 640)])
    pltpu.sync_copy(s_sh.at[pl.ds(r0, 640)], s_out.at[c, pl.ds(r0, 640)])


_edge_pass = functools.partial(
    pl.kernel,
    out_type=[
        jax.ShapeDtypeStruct((2, N, ACCW), jnp.float32),
        jax.ShapeDtypeStruct((2, N), jnp.float32),
    ],
    mesh=plsc.VectorSubcoreMesh(core_axis_name="c", subcore_axis_name="s"),
    compiler_params=pltpu.CompilerParams(
        needs_layout_passes=False, use_tc_tiling_on_sc=False),
    scratch_types=[
        pltpu.VMEM((EPT,), jnp.int32),
        pltpu.VMEM((CH, WROW), jnp.float32),
        pltpu.VMEM((CH, WROW), jnp.float32),
        pltpu.VMEM((CH, ACCW), jnp.float32),
        pltpu.VMEM((CH, ACCW), jnp.float32),
        pltpu.VMEM((CH,), jnp.float32),
        pltpu.VMEM((CH,), jnp.float32),
        pltpu.VMEM((CH,), jnp.float32),
        pltpu.VMEM((CH,), jnp.float32),
        pltpu.VMEM((CH,), jnp.int32),
        pltpu.VMEM((CH,), jnp.int32),
        pltpu.VMEM((CH,), jnp.int32),
        pltpu.VMEM((CH,), jnp.int32),
        pltpu.VMEM((CH,), jnp.int32),
        pltpu.VMEM((CH,), jnp.int32),
        pltpu.VMEM((CH,), jnp.float32),
        pltpu.VMEM((CH,), jnp.float32),
        pltpu.VMEM((128,), jnp.float32),
        pltpu.VMEM_SHARED((NPAD, ACCW), jnp.float32),
        pltpu.VMEM_SHARED((NPAD,), jnp.float32),
        pltpu.SemaphoreType.DMA,
        pltpu.SemaphoreType.DMA,
        pltpu.SemaphoreType.DMA,
        pltpu.SemaphoreType.DMA,
    ],
)(_edge_body)


def _tc_a_body(x_ref, w_ref, avs_ref, avd_ref, hpad_ref, as_ref, ad_ref):
    h = jnp.dot(x_ref[...], w_ref[...], preferred_element_type=jnp.float32)
    hpad_ref[:, :D] = h
    pad = (lax.broadcasted_iota(jnp.int32, (BN, WROW - D), 1) == 0)
    hpad_ref[:, D:] = pad.astype(jnp.float32)
    as_ref[...] = jnp.dot(h, avs_ref[...], preferred_element_type=jnp.float32)
    ad_ref[...] = jnp.dot(h, avd_ref[...], preferred_element_type=jnp.float32)


def _tc_a(x, w, avs, avd):
    return pl.pallas_call(
        _tc_a_body,
        grid=(NBLK,),
        in_specs=[
            pl.BlockSpec((BN, D), lambda i: (i, 0)),
            pl.BlockSpec((D, D), lambda i: (0, 0)),
            pl.BlockSpec((D, 1), lambda i: (0, 0)),
            pl.BlockSpec((D, 1), lambda i: (0, 0)),
        ],
        out_specs=[
            pl.BlockSpec((BN, WROW), lambda i: (i, 0)),
            pl.BlockSpec((BN, 1), lambda i: (i, 0)),
            pl.BlockSpec((BN, 1), lambda i: (i, 0)),
        ],
        out_shape=[
            jax.ShapeDtypeStruct((N, WROW), jnp.float32),
            jax.ShapeDtypeStruct((N, 1), jnp.float32),
            jax.ShapeDtypeStruct((N, 1), jnp.float32),
        ],
    )(x, w, avs, avd)


def _combine(part_ref, s0_ref, s1_ref, b_ref):
    den = s0_ref[...] + s1_ref[...] + 1e-16
    return (part_ref[0] + part_ref[1]) / den + b_ref[...]


def _tc_b_body(part_ref, s0_ref, s1_ref, b_ref, w_ref, avs_ref, avd_ref,
               hpad_ref, as_ref, ad_ref):
    feats = _combine(part_ref, s0_ref, s1_ref, b_ref)
    h = jnp.dot(feats, w_ref[...], preferred_element_type=jnp.float32)
    hpad_ref[:, :D] = h
    pad = (lax.broadcasted_iota(jnp.int32, (BN, WROW - D), 1) == 0)
    hpad_ref[:, D:] = pad.astype(jnp.float32)
    as_ref[...] = jnp.dot(h, avs_ref[...], preferred_element_type=jnp.float32)
    ad_ref[...] = jnp.dot(h, avd_ref[...], preferred_element_type=jnp.float32)


def _tc_b(part, s0, s1, b, w, avs, avd):
    return pl.pallas_call(
        _tc_b_body,
        grid=(NBLK,),
        in_specs=[
            pl.BlockSpec((2, BN, ACCW), lambda i: (0, i, 0)),
            pl.BlockSpec((BN, 1), lambda i: (i, 0)),
            pl.BlockSpec((BN, 1), lambda i: (i, 0)),
            pl.BlockSpec((1, D), lambda i: (0, 0)),
            pl.BlockSpec((D, D), lambda i: (0, 0)),
            pl.BlockSpec((D, 1), lambda i: (0, 0)),
            pl.BlockSpec((D, 1), lambda i: (0, 0)),
        ],
        out_specs=[
            pl.BlockSpec((BN, WROW), lambda i: (i, 0)),
            pl.BlockSpec((BN, 1), lambda i: (i, 0)),
            pl.BlockSpec((BN, 1), lambda i: (i, 0)),
        ],
        out_shape=[
            jax.ShapeDtypeStruct((N, WROW), jnp.float32),
            jax.ShapeDtypeStruct((N, 1), jnp.float32),
            jax.ShapeDtypeStruct((N, 1), jnp.float32),
        ],
    )(part, s0, s1, b, w, avs, avd)


def _tc_c_body(part_ref, s0_ref, s1_ref, b_ref, batch_ref, out_ref, sums, cnt):
    i = pl.program_id(0)

    @pl.when(i == 0)
    def _():
        sums[...] = jnp.zeros_like(sums)
        cnt[...] = jnp.zeros_like(cnt)

    feats = _combine(part_ref, s0_ref, s1_ref, b_ref)
    bblk = batch_ref[0, 0, :]
    oh = (bblk[None, :] == lax.broadcasted_iota(jnp.int32, (G, BN), 0))
    oh = oh.astype(jnp.float32)
    sums[...] += jnp.dot(oh, feats, preferred_element_type=jnp.float32)
    cnt[...] += jnp.sum(oh, axis=1, keepdims=True)

    @pl.when(i == NBLK - 1)
    def _():
        out_ref[...] = sums[...] / jnp.maximum(cnt[...], 1.0)


def _tc_c(part, s0, s1, b, batch3):
    return pl.pallas_call(
        _tc_c_body,
        grid=(NBLK,),
        in_specs=[
            pl.BlockSpec((2, BN, ACCW), lambda i: (0, i, 0)),
            pl.BlockSpec((BN, 1), lambda i: (i, 0)),
            pl.BlockSpec((BN, 1), lambda i: (i, 0)),
            pl.BlockSpec((1, D), lambda i: (0, 0)),
            pl.BlockSpec((1, 1, BN), lambda i: (i, 0, 0)),
        ],
        out_specs=pl.BlockSpec((G, D), lambda i: (0, 0)),
        out_shape=jax.ShapeDtypeStruct((G, D), jnp.float32),
        scratch_shapes=[
            pltpu.VMEM((G, D), jnp.float32),
            pltpu.VMEM((G, 1), jnp.float32),
        ],
    )(part, s0, s1, b, batch3)


def kernel(x, edge_index, batch, W1, att_src1, att_dst1, b1,
           W2, att_src2, att_dst2, b2):
    srcp = jnp.concatenate(
        [edge_index[0], jnp.zeros((EPAD - E,), jnp.int32)])
    dstp = jnp.concatenate(
        [edge_index[1], jnp.full((EPAD - E,), N, jnp.int32)])
    packed = jnp.bitwise_or(srcp, jnp.left_shift(dstp, 14))
    zpad16 = jnp.zeros((16,), jnp.float32)

    hpad1, as1, ad1 = _tc_a(x, W1, att_src1.reshape(D, 1),
                            att_dst1.reshape(D, 1))
    acc1, s1 = _edge_pass(hpad1, as1.reshape(N),
                          jnp.concatenate([ad1.reshape(N), zpad16]), packed)
    hpad2, as2, ad2 = _tc_b(acc1, s1[0].reshape(N, 1), s1[1].reshape(N, 1),
                            b1.reshape(1, D), W2,
                            att_src2.reshape(D, 1), att_dst2.reshape(D, 1))
    acc2, s2 = _edge_pass(hpad2, as2.reshape(N),
                          jnp.concatenate([ad2.reshape(N), zpad16]), packed)
    return _tc_c(acc2, s2[0].reshape(N, 1), s2[1].reshape(N, 1),
                 b2.reshape(1, D), batch.reshape(NBLK, 1, BN))
